# baseline (speedup 1.0000x reference)
import functools

import jax
import jax.numpy as jnp
from jax import lax
from jax.experimental import pallas as pl
from jax.experimental.pallas import tpu as pltpu
from jax.experimental.pallas import tpu_sc as plsc

NW = 32
CHUNK = 128


def _gather_kernel(n_chunks, chunk, d, table_hbm, idx_hbm, out_hbm,
                   idx_v, rows_v, gsem):
    wid = lax.axis_index("s") * 2 + lax.axis_index("c")
    base = wid * (n_chunks * chunk)
    pltpu.sync_copy(idx_hbm.at[wid], idx_v)

    def body(j, carry):
        pltpu.async_copy(table_hbm.at[idx_v.at[j]], rows_v, gsem).wait()
        pltpu.sync_copy(rows_v, out_hbm.at[pl.ds(base + j * chunk, chunk)])
        return carry

    lax.fori_loop(0, n_chunks, body, 0, unroll=False)


def kernel(characters, tokens, table):
    B, L = tokens.shape
    V, D = table.shape
    N = B * L
    n_per_w = N // NW
    n_chunks = n_per_w // CHUNK

    idx = tokens.reshape(NW, n_chunks, CHUNK).astype(jnp.int32)

    mesh = plsc.VectorSubcoreMesh(core_axis_name="c", subcore_axis_name="s")
    run = functools.partial(
        pl.kernel,
        out_type=jax.ShapeDtypeStruct((N, D), jnp.float32),
        mesh=mesh,
        compiler_params=pltpu.CompilerParams(use_tc_tiling_on_sc=False),
        scratch_types=[
            pltpu.VMEM((n_chunks, CHUNK), jnp.int32),
            pltpu.VMEM((CHUNK, D), jnp.float32),
            pltpu.SemaphoreType.DMA,
        ],
    )(functools.partial(_gather_kernel, n_chunks, CHUNK, D))

    out = run(table, idx)
    return out.reshape(B, L, D)


# speedup vs baseline: 1.1175x; 1.1175x over previous
import functools

import jax
import jax.numpy as jnp
from jax import lax
from jax.experimental import pallas as pl
from jax.experimental.pallas import tpu as pltpu
from jax.experimental.pallas import tpu_sc as plsc

NW = 32
CHUNK = 128
K = 2
NSET = 5


def _gather_kernel(n_chunks, table_hbm, idx_hbm, out_hbm,
                   idx_v, rows_v, gsem, wsem):
    n_groups = n_chunks // K
    wid = lax.axis_index("s") * 2 + lax.axis_index("c")
    base = wid * (n_chunks * CHUNK)
    pltpu.sync_copy(idx_hbm.at[wid], idx_v)

    def gather_copy(g, p, b):
        return pltpu.make_async_copy(
            table_hbm.at[idx_v.at[g * K + b]], rows_v.at[p, b], gsem)

    def write_copy(g, p, b):
        dst = out_hbm.at[pl.ds(base + (g * K + b) * CHUNK, CHUNK)]
        return pltpu.make_async_copy(rows_v.at[p, b], dst, wsem)

    def start_gathers(g, p):
        for b in range(K):
            gather_copy(g, p, b).start()

    start_gathers(0, 0)
    start_gathers(1, 1)

    @pl.loop(0, n_groups, step=NSET)
    def _(g0):
        for p in range(NSET):
            g = g0 + p
            for b in range(K):
                gather_copy(g, p, b).wait()
            for b in range(K):
                write_copy(g, p, b).start()
            pw = (p + 2) % NSET

            @pl.when(g >= 3)
            def _():
                for b in range(K):
                    write_copy(g - 3, pw, b).wait()

            @pl.when(g + 2 < n_groups)
            def _():
                start_gathers(g + 2, pw)

    for g in (n_groups - 3, n_groups - 2, n_groups - 1):
        for b in range(K):
            write_copy(g, g % NSET, b).wait()


def kernel(characters, tokens, table):
    B, L = tokens.shape
    V, D = table.shape
    N = B * L
    n_per_w = N // NW
    n_chunks = n_per_w // CHUNK

    idx = tokens.reshape(NW, n_chunks, CHUNK).astype(jnp.int32)

    mesh = plsc.VectorSubcoreMesh(core_axis_name="c", subcore_axis_name="s")
    run = functools.partial(
        pl.kernel,
        out_type=jax.ShapeDtypeStruct((N, D), jnp.float32),
        mesh=mesh,
        compiler_params=pltpu.CompilerParams(use_tc_tiling_on_sc=False),
        scratch_types=[
            pltpu.VMEM((n_chunks, CHUNK), jnp.int32),
            pltpu.VMEM((NSET, K, CHUNK, D), jnp.float32),
            pltpu.SemaphoreType.DMA,
            pltpu.SemaphoreType.DMA,
        ],
    )(functools.partial(_gather_kernel, n_chunks))

    out = run(table, idx)
    return out.reshape(B, L, D)
